# Initial kernel scaffold; baseline (speedup 1.0000x reference)
import jax
import jax.numpy as jnp
from jax.experimental import pallas as pl


def kernel(encoded_sents, mask, keys, U, V, W):
    raise NotImplementedError("write your pallas kernel here")


# speedup vs baseline: 2.9774x; 2.9774x over previous
import jax
import jax.numpy as jnp
from jax.experimental import pallas as pl

B, S, K, D = 1024, 20, 100, 32
BT = 128
NT = B // BT
C = K * BT


def _entity_kernel(x_ref, m_ref, keys_ref, Ut_ref, Vt_ref, Wt_ref, out_ref):
    KT = keys_ref[0]
    Ut = Ut_ref[...]
    Vt = Vt_ref[...]
    Wt = Wt_ref[...]

    kV = jnp.dot(Vt, KT, preferred_element_type=jnp.float32)

    def step(t, H):
        x = x_ref[t]
        m = m_ref[t]
        xT = jnp.concatenate([x] * K, axis=1)
        xW = jnp.dot(Wt, x, preferred_element_type=jnp.float32)
        xWT = jnp.concatenate([xW] * K, axis=1)
        mT = jnp.concatenate([m] * K, axis=1)

        g = jax.nn.sigmoid(jnp.sum(xT * (H + KT), axis=0, keepdims=True))
        hU = jnp.dot(Ut, H, preferred_element_type=jnp.float32)
        h_tilda = jax.nn.relu(hU + kV + xWT)
        upd = H + g * h_tilda
        sq = jnp.sum(upd * upd, axis=0, keepdims=True)
        upd = upd * jax.lax.rsqrt(jnp.maximum(sq, 1e-12))
        return H + mT * (upd - H)

    H0 = jnp.zeros((D, C), dtype=jnp.float32)
    out_ref[0] = jax.lax.fori_loop(0, S, step, H0)


@jax.jit
def kernel(encoded_sents, mask, keys, U, V, W):
    x_all = jnp.transpose(encoded_sents, (1, 2, 0))
    m_all = jnp.swapaxes(mask, 0, 1).astype(jnp.float32)[:, None, :]
    keysR = jnp.transpose(keys, (2, 1, 0))
    keysR = keysR.reshape(D, K, NT, BT).transpose(2, 0, 1, 3).reshape(NT, D, C)

    hT = pl.pallas_call(
        _entity_kernel,
        grid=(NT,),
        in_specs=[
            pl.BlockSpec((S, D, BT), lambda i: (0, 0, i)),
            pl.BlockSpec((S, 1, BT), lambda i: (0, 0, i)),
            pl.BlockSpec((1, D, C), lambda i: (i, 0, 0)),
            pl.BlockSpec((D, D), lambda i: (0, 0)),
            pl.BlockSpec((D, D), lambda i: (0, 0)),
            pl.BlockSpec((D, D), lambda i: (0, 0)),
        ],
        out_specs=pl.BlockSpec((1, D, C), lambda i: (i, 0, 0)),
        out_shape=jax.ShapeDtypeStruct((NT, D, C), jnp.float32),
    )(x_all, m_all, keysR, U.T, V.T, W.T)

    out = hT.reshape(NT, D, K, BT).transpose(0, 3, 2, 1).reshape(B, K, D)
    return out


# speedup vs baseline: 3.5290x; 1.1853x over previous
import jax
import jax.numpy as jnp
from jax.experimental import pallas as pl
from jax.experimental.pallas import tpu as pltpu

B, S, K, D = 1024, 20, 100, 32
BT = 128
NT = B // BT
C = K * BT
CH = 256
NC = C // CH


def _entity_kernel(x_ref, m_ref, keys_ref, Ut_ref, Vt_ref, Wt_ref, out_ref,
                   kv_ref, hu_ref, hb_ref):
    Ut = Ut_ref[...]
    Wt = Wt_ref[...]

    kv_ref[...] = jnp.dot(Vt_ref[...], keys_ref[0],
                          preferred_element_type=jnp.float32)
    out_ref[0] = jnp.zeros((D, C), dtype=jnp.float32)

    def substep(t, src, dst):
        x = x_ref[t]
        m = m_ref[t]
        xW = jnp.dot(Wt, x, preferred_element_type=jnp.float32)
        x2 = jnp.concatenate([x, x], axis=1)
        m2 = jnp.concatenate([m, m], axis=1)
        xw2 = jnp.concatenate([xW, xW], axis=1)
        Hfull = src[0] if src is out_ref else src[...]
        hu_ref[...] = jnp.dot(Ut, Hfull, preferred_element_type=jnp.float32)
        for c in range(NC):
            sl = slice(c * CH, (c + 1) * CH)
            if src is out_ref:
                Hc = src[0, :, sl]
            else:
                Hc = src[:, sl]
            Kc = keys_ref[0, :, sl]
            g = jax.nn.sigmoid(
                jnp.sum(x2 * (Hc + Kc), axis=0, keepdims=True))
            ht = jnp.maximum(hu_ref[:, sl] + kv_ref[:, sl] + xw2, 0.0)
            u = Hc + g * ht
            sq = jnp.sum(u * u, axis=0, keepdims=True)
            scale = jax.lax.rsqrt(jnp.maximum(sq, 1e-12))
            res = Hc * (1.0 - m2) + (m2 * scale) * u
            if dst is out_ref:
                dst[0, :, sl] = res
            else:
                dst[:, sl] = res

    def double_step(i, carry):
        substep(2 * i, out_ref, hb_ref)
        substep(2 * i + 1, hb_ref, out_ref)
        return carry

    jax.lax.fori_loop(0, S // 2, double_step, 0)


@jax.jit
def kernel(encoded_sents, mask, keys, U, V, W):
    x_all = jnp.transpose(encoded_sents, (1, 2, 0))
    m_all = jnp.swapaxes(mask, 0, 1).astype(jnp.float32)[:, None, :]
    keysR = jnp.transpose(keys, (2, 1, 0))
    keysR = keysR.reshape(D, K, NT, BT).transpose(2, 0, 1, 3).reshape(NT, D, C)

    hT = pl.pallas_call(
        _entity_kernel,
        grid=(NT,),
        in_specs=[
            pl.BlockSpec((S, D, BT), lambda i: (0, 0, i)),
            pl.BlockSpec((S, 1, BT), lambda i: (0, 0, i)),
            pl.BlockSpec((1, D, C), lambda i: (i, 0, 0)),
            pl.BlockSpec((D, D), lambda i: (0, 0)),
            pl.BlockSpec((D, D), lambda i: (0, 0)),
            pl.BlockSpec((D, D), lambda i: (0, 0)),
        ],
        out_specs=pl.BlockSpec((1, D, C), lambda i: (i, 0, 0)),
        out_shape=jax.ShapeDtypeStruct((NT, D, C), jnp.float32),
        scratch_shapes=[
            pltpu.VMEM((D, C), jnp.float32),
            pltpu.VMEM((D, C), jnp.float32),
            pltpu.VMEM((D, C), jnp.float32),
        ],
    )(x_all, m_all, keysR, U.T, V.T, W.T)

    out = hT.reshape(NT, D, K, BT).transpose(0, 3, 2, 1).reshape(B, K, D)
    return out


# speedup vs baseline: 3.6531x; 1.0352x over previous
import jax
import jax.numpy as jnp
from jax.experimental import pallas as pl
from jax.experimental.pallas import tpu as pltpu

B, S, K, D = 1024, 20, 100, 32
BT = 128
NT = B // BT
C = K * BT
CH = 256
NC = C // CH


def _entity_kernel(x_ref, m_ref, keys_ref, Ut_ref, Vt_ref, Wt_ref, out_ref,
                   kv_ref, hu_ref, hb_ref):
    Ut = Ut_ref[...]
    Wt = Wt_ref[...]
    ones_row = jnp.ones((1, D), dtype=jnp.float32)

    kv_ref[...] = jnp.dot(Vt_ref[...], keys_ref[0],
                          preferred_element_type=jnp.float32)
    out_ref[0] = jnp.zeros((D, C), dtype=jnp.float32)

    def substep(t, src, dst):
        x = x_ref[t]
        m = m_ref[t]
        xW = jnp.dot(Wt, x, preferred_element_type=jnp.float32)
        rep = CH // BT
        x2 = jnp.concatenate([x] * rep, axis=1)
        m2 = jnp.concatenate([m] * rep, axis=1)
        notm2 = 1.0 - m2
        xw2 = jnp.concatenate([xW] * rep, axis=1)
        Hfull = src[0] if src is out_ref else src[...]
        hu_ref[...] = jnp.dot(Ut, Hfull,
                              preferred_element_type=jnp.float32) + kv_ref[...]
        for c in range(NC):
            sl = slice(c * CH, (c + 1) * CH)
            if src is out_ref:
                Hc = src[0, :, sl]
            else:
                Hc = src[:, sl]
            Kc = keys_ref[0, :, sl]
            g = jax.nn.sigmoid(
                jnp.dot(ones_row, x2 * (Hc + Kc),
                        preferred_element_type=jnp.float32))
            ht = jnp.maximum(hu_ref[:, sl] + xw2, 0.0)
            u = Hc + g * ht
            sq = jnp.dot(ones_row, u * u,
                         preferred_element_type=jnp.float32)
            scale = jax.lax.rsqrt(jnp.maximum(sq, 1e-12))
            res = Hc * notm2 + (m2 * scale) * u
            if dst is out_ref:
                dst[0, :, sl] = res
            else:
                dst[:, sl] = res

    def double_step(i, carry):
        substep(2 * i, out_ref, hb_ref)
        substep(2 * i + 1, hb_ref, out_ref)
        return carry

    jax.lax.fori_loop(0, S // 2, double_step, 0)


@jax.jit
def kernel(encoded_sents, mask, keys, U, V, W):
    x_all = jnp.transpose(encoded_sents, (1, 2, 0))
    m_all = jnp.swapaxes(mask, 0, 1).astype(jnp.float32)[:, None, :]
    keysR = jnp.transpose(keys, (2, 1, 0))
    keysR = keysR.reshape(D, K, NT, BT).transpose(2, 0, 1, 3).reshape(NT, D, C)

    hT = pl.pallas_call(
        _entity_kernel,
        grid=(NT,),
        in_specs=[
            pl.BlockSpec((S, D, BT), lambda i: (0, 0, i)),
            pl.BlockSpec((S, 1, BT), lambda i: (0, 0, i)),
            pl.BlockSpec((1, D, C), lambda i: (i, 0, 0)),
            pl.BlockSpec((D, D), lambda i: (0, 0)),
            pl.BlockSpec((D, D), lambda i: (0, 0)),
            pl.BlockSpec((D, D), lambda i: (0, 0)),
        ],
        out_specs=pl.BlockSpec((1, D, C), lambda i: (i, 0, 0)),
        out_shape=jax.ShapeDtypeStruct((NT, D, C), jnp.float32),
        scratch_shapes=[
            pltpu.VMEM((D, C), jnp.float32),
            pltpu.VMEM((D, C), jnp.float32),
            pltpu.VMEM((D, C), jnp.float32),
        ],
    )(x_all, m_all, keysR, U.T, V.T, W.T)

    out = hT.reshape(NT, D, K, BT).transpose(0, 3, 2, 1).reshape(B, K, D)
    return out
